# baseline (speedup 1.0000x reference)
import functools

import jax
import jax.numpy as jnp
from jax import lax
from jax.experimental import pallas as pl
from jax.experimental.pallas import tpu as pltpu
from jax.experimental.pallas import tpu_sc as plsc

N_CORES = 2
N_SUBCORES = 16
NW = N_CORES * N_SUBCORES
L = 16
CHUNK = 4096


def _build_sc_kernel(n_sites, npix, width_s):
    mesh = plsc.VectorSubcoreMesh(
        core_axis_name="c", subcore_axis_name="s",
        num_cores=N_CORES, num_subcores=N_SUBCORES)
    per_w = npix // NW
    n_chunks = per_w // CHUNK
    groups = CHUNK // L
    tiles_w = width_s // 128

    @functools.partial(
        pl.kernel,
        out_type=jax.ShapeDtypeStruct((npix * 3,), jnp.float32),
        mesh=mesh,
        scratch_types=[
            pltpu.VMEM((n_sites * 4,), jnp.float32),
            [pltpu.VMEM((CHUNK,), jnp.int32) for _ in range(2)],
            [pltpu.VMEM((CHUNK,), jnp.int32) for _ in range(2)],
            [[pltpu.VMEM((CHUNK,), jnp.float32) for _ in range(3)]
             for _ in range(2)],
            pltpu.VMEM((3 * L,), jnp.float32),
            pltpu.VMEM((width_s,), jnp.float32),
            pltpu.VMEM((npix // width_s,), jnp.float32),
            pltpu.SemaphoreType.DMA,
            [pltpu.SemaphoreType.DMA for _ in range(2)],
            [pltpu.SemaphoreType.DMA for _ in range(2)],
        ],
        compiler_params=pltpu.CompilerParams(use_tc_tiling_on_sc=False,
                                             needs_layout_passes=False),
    )
    def sad_sc(sites_hbm, c0_hbm, c1_hbm, par_hbm, out_hbm,
               table_v, c0_v, c1_v, out_v, par_v, lut_u, lut_v,
               tab_sem, in_sems, out_sems):
        wid = lax.axis_index("s") * N_CORES + lax.axis_index("c")
        base_w = wid * per_w

        def start_in(ci):
            b = ci % 2
            base = base_w + ci * CHUNK
            h0 = pltpu.async_copy(c0_hbm.at[pl.ds(base, CHUNK)], c0_v[b],
                                  in_sems[b])
            h1 = pltpu.async_copy(c1_hbm.at[pl.ds(base, CHUNK)], c1_v[b],
                                  in_sems[b])
            return (h0, h1)

        tab_h = pltpu.async_copy(sites_hbm, table_v, tab_sem)
        in_h = [None] * n_chunks
        out_h = [None] * n_chunks
        in_h[0] = start_in(0)
        pltpu.sync_copy(par_hbm, par_v)
        inv_w = par_v[pl.ds(0, L)]
        inv_h = par_v[pl.ds(L, L)]
        scale = par_v[pl.ds(2 * L, L)]
        iota = lax.iota(jnp.int32, L)

        @plsc.parallel_loop(0, width_s // L, 1, unroll=4)
        def bu(i):
            xv = (i * L + iota).astype(jnp.float32)
            lut_u[pl.ds(i * L, L)] = (xv + 0.5) * inv_w * 2.0

        @plsc.parallel_loop(0, npix // width_s // L, 1, unroll=4)
        def bv(i):
            yv = (i * L + iota).astype(jnp.float32)
            lut_v[pl.ds(i * L, L)] = (yv + 0.5) * inv_h * 2.0

        tab_h.wait()

        for ci in range(n_chunks):
            b = ci % 2
            base = base_w + ci * CHUNK
            for h in in_h[ci]:
                h.wait()
            if ci + 1 < n_chunks:
                in_h[ci + 1] = start_in(ci + 1)
            if ci >= 2:
                for h in out_h[ci - 2]:
                    h.wait()
            c0b, c1b, outb = c0_v[b], c1_v[b], out_v[b]

            @plsc.parallel_loop(0, groups // 2, 1, unroll=3)
            def grp(g2):
                off0 = g2 * (2 * L)
                w_off = base + off0
                wo = w_off & (8 * width_s - 1)
                cc = lax.shift_right_logical(wo, 10)
                r = lax.shift_right_logical(wo, 7) & 7
                l0 = wo & 127
                t_glob = lax.shift_right_logical(w_off, 13)
                yidx = jnp.full((L,), t_glob * 8 + r, jnp.int32)
                py2 = plsc.load_gather(lut_v, [yidx])
                s0 = cc * 128 + l0
                for sub in range(2):
                    off = off0 + sub * L
                    idx0 = c0b[pl.ds(off, L)]
                    idx1 = c1b[pl.ds(off, L)]
                    x0 = plsc.load_gather(table_v, [idx0])
                    y0 = plsc.load_gather(table_v, [idx0 + n_sites])
                    x1 = plsc.load_gather(table_v, [idx1])
                    y1 = plsc.load_gather(table_v, [idx1 + n_sites])
                    px2 = lut_u[pl.ds(s0 + sub * L, L)]
                    t = ((x0 - x1) * (px2 - x0 - x1)
                         + (y0 - y1) * (py2 - y0 - y1)) * scale
                    w = 1.0 / (1.0 + jnp.exp(-t))
                    rg0 = plsc.load_gather(table_v, [idx0 + 2 * n_sites])
                    rg1 = plsc.load_gather(table_v, [idx1 + 2 * n_sites])
                    b0 = plsc.load_gather(table_v, [idx0 + 3 * n_sites])
                    b1 = plsc.load_gather(table_v, [idx1 + 3 * n_sites])
                    wpair = plsc.pack(w, w,
                                      format=plsc.PackFormat.INTERLEAVED)
                    bf0 = plsc.bitcast(rg0, jnp.bfloat16)
                    bf1 = plsc.bitcast(rg1, jnp.bfloat16)
                    obf = bf1 + wpair * (bf0 - bf1)
                    ou = plsc.bitcast(obf, jnp.uint32)
                    himask = jnp.full((L,), 0xFFFF0000, jnp.uint32)
                    outb[0][pl.ds(off, L)] = plsc.bitcast(ou << 16,
                                                          jnp.float32)
                    outb[1][pl.ds(off, L)] = plsc.bitcast(ou & himask,
                                                          jnp.float32)
                    outb[2][pl.ds(off, L)] = b1 + w * (b0 - b1)

            out_h[ci] = tuple(
                pltpu.async_copy(outb[c],
                                 out_hbm.at[pl.ds(c * npix + base, CHUNK)],
                                 out_sems[b])
                for c in range(3))

        for ci in (n_chunks - 2, n_chunks - 1):
            for h in out_h[ci]:
                h.wait()

    return sad_sc


def kernel(sites, cand0, cand1, width, height, inv_scale_sq):
    height_s, width_s = cand0.shape
    n_sites = sites.shape[0]
    npix = height_s * width_s

    width_f = jnp.asarray(width, dtype=jnp.float32)
    height_f = jnp.asarray(height, dtype=jnp.float32)
    scale_f = jnp.asarray(inv_scale_sq, dtype=jnp.float32)
    params = jnp.concatenate([
        jnp.broadcast_to(1.0 / width_f, (L,)),
        jnp.broadcast_to(1.0 / height_f, (L,)),
        jnp.broadcast_to(scale_f, (L,)),
    ]).astype(jnp.float32)

    def tile_order(c):
        c4 = c.reshape(height_s // 8, 8, width_s // 128, 128)
        return c4.transpose(0, 2, 1, 3).reshape(npix)

    r16 = lax.bitcast_convert_type(
        sites[:, 2].astype(jnp.bfloat16), jnp.uint16).astype(jnp.uint32)
    g16 = lax.bitcast_convert_type(
        sites[:, 3].astype(jnp.bfloat16), jnp.uint16).astype(jnp.uint32)
    rg = lax.bitcast_convert_type(r16 | (g16 << 16), jnp.float32)
    sites_cols = jnp.concatenate(
        [sites[:, 0], sites[:, 1], rg, sites[:, 4]])
    sad_sc = _build_sc_kernel(n_sites, npix, width_s)
    out_flat = sad_sc(sites_cols, tile_order(cand0), tile_order(cand1),
                      params)
    out5 = out_flat.reshape(3, height_s // 8, width_s // 128, 8, 128)
    return out5.transpose(1, 3, 2, 4, 0).reshape(height_s, width_s, 3)


# speedup vs baseline: 1.0781x; 1.0781x over previous
import functools

import jax
import jax.numpy as jnp
from jax import lax
from jax.experimental import pallas as pl
from jax.experimental.pallas import tpu as pltpu
from jax.experimental.pallas import tpu_sc as plsc

N_CORES = 2
N_SUBCORES = 16
NW = N_CORES * N_SUBCORES
L = 16
CHUNK = 4096


def _build_sc_kernel(n_sites, npix, width_s):
    mesh = plsc.VectorSubcoreMesh(
        core_axis_name="c", subcore_axis_name="s",
        num_cores=N_CORES, num_subcores=N_SUBCORES)
    per_w = npix // NW
    n_chunks = per_w // CHUNK
    groups = CHUNK // L
    tiles_w = width_s // 128

    @functools.partial(
        pl.kernel,
        out_type=jax.ShapeDtypeStruct((npix * 3,), jnp.float32),
        mesh=mesh,
        scratch_types=[
            pltpu.VMEM((n_sites * 4,), jnp.float32),
            [pltpu.VMEM((CHUNK,), jnp.int32) for _ in range(2)],
            [pltpu.VMEM((CHUNK,), jnp.int32) for _ in range(2)],
            [[pltpu.VMEM((CHUNK,), jnp.float32) for _ in range(3)]
             for _ in range(2)],
            pltpu.VMEM((3 * L,), jnp.float32),
            pltpu.VMEM((width_s,), jnp.float32),
            pltpu.VMEM((npix // width_s,), jnp.float32),
            pltpu.SemaphoreType.DMA,
            [pltpu.SemaphoreType.DMA for _ in range(2)],
            [pltpu.SemaphoreType.DMA for _ in range(2)],
        ],
        compiler_params=pltpu.CompilerParams(use_tc_tiling_on_sc=False,
                                             needs_layout_passes=False),
    )
    def sad_sc(sites_hbm, c0_hbm, c1_hbm, par_hbm, out_hbm,
               table_v, c0_v, c1_v, out_v, par_v, lut_u, lut_v,
               tab_sem, in_sems, out_sems):
        wid = lax.axis_index("s") * N_CORES + lax.axis_index("c")
        base_w = wid * per_w

        def start_in(ci):
            b = ci % 2
            base = base_w + ci * CHUNK
            h0 = pltpu.async_copy(c0_hbm.at[pl.ds(base, CHUNK)], c0_v[b],
                                  in_sems[b])
            h1 = pltpu.async_copy(c1_hbm.at[pl.ds(base, CHUNK)], c1_v[b],
                                  in_sems[b])
            return (h0, h1)

        tab_h = pltpu.async_copy(sites_hbm, table_v, tab_sem)
        in_h = [None] * n_chunks
        out_h = [None] * n_chunks
        in_h[0] = start_in(0)
        pltpu.sync_copy(par_hbm, par_v)
        inv_w = par_v[pl.ds(0, L)]
        inv_h = par_v[pl.ds(L, L)]
        scale = par_v[pl.ds(2 * L, L)]
        iota = lax.iota(jnp.int32, L)

        @plsc.parallel_loop(0, width_s // L, 1, unroll=4)
        def bu(i):
            xv = (i * L + iota).astype(jnp.float32)
            lut_u[pl.ds(i * L, L)] = (xv + 0.5) * inv_w * 2.0

        @plsc.parallel_loop(0, npix // width_s // L, 1, unroll=4)
        def bv(i):
            yv = (i * L + iota).astype(jnp.float32)
            lut_v[pl.ds(i * L, L)] = (yv + 0.5) * inv_h * 2.0

        tab_h.wait()

        for ci in range(n_chunks):
            b = ci % 2
            base = base_w + ci * CHUNK
            for h in in_h[ci]:
                h.wait()
            if ci + 1 < n_chunks:
                in_h[ci + 1] = start_in(ci + 1)
            if ci >= 2:
                for h in out_h[ci - 2]:
                    h.wait()
            c0b, c1b, outb = c0_v[b], c1_v[b], out_v[b]

            @plsc.parallel_loop(0, groups, 1, unroll=4)
            def grp(g):
                off = g * L
                w_off = base + off
                wo = w_off & (8 * width_s - 1)
                cc = lax.shift_right_logical(wo, 10)
                r = lax.shift_right_logical(wo, 7) & 7
                l0 = wo & 127
                t_glob = lax.shift_right_logical(w_off, 13)
                idx0 = c0b[pl.ds(off, L)]
                idx1 = c1b[pl.ds(off, L)]
                x0 = plsc.load_gather(table_v, [idx0])
                y0 = plsc.load_gather(table_v, [idx0 + n_sites])
                x1 = plsc.load_gather(table_v, [idx1])
                y1 = plsc.load_gather(table_v, [idx1 + n_sites])
                px2 = lut_u[pl.ds(cc * 128 + l0, L)]
                yidx = jnp.full((L,), t_glob * 8 + r, jnp.int32)
                py2 = plsc.load_gather(lut_v, [yidx])
                t = ((x0 - x1) * (px2 - x0 - x1)
                     + (y0 - y1) * (py2 - y0 - y1)) * scale
                w = 1.0 / (1.0 + jnp.exp(-t))
                rg0 = plsc.load_gather(table_v, [idx0 + 2 * n_sites])
                rg1 = plsc.load_gather(table_v, [idx1 + 2 * n_sites])
                b0 = plsc.load_gather(table_v, [idx0 + 3 * n_sites])
                b1 = plsc.load_gather(table_v, [idx1 + 3 * n_sites])
                wpair = plsc.pack(w, w, format=plsc.PackFormat.INTERLEAVED)
                bf0 = plsc.bitcast(rg0, jnp.bfloat16)
                bf1 = plsc.bitcast(rg1, jnp.bfloat16)
                obf = bf1 + wpair * (bf0 - bf1)
                ou = plsc.bitcast(obf, jnp.uint32)
                himask = jnp.full((L,), 0xFFFF0000, jnp.uint32)
                outb[0][pl.ds(off, L)] = plsc.bitcast(ou << 16, jnp.float32)
                outb[1][pl.ds(off, L)] = plsc.bitcast(ou & himask,
                                                      jnp.float32)
                outb[2][pl.ds(off, L)] = b1 + w * (b0 - b1)

            out_h[ci] = tuple(
                pltpu.async_copy(outb[c],
                                 out_hbm.at[pl.ds(c * npix + base, CHUNK)],
                                 out_sems[b])
                for c in range(3))

        for ci in (n_chunks - 2, n_chunks - 1):
            for h in out_h[ci]:
                h.wait()

    return sad_sc


def kernel(sites, cand0, cand1, width, height, inv_scale_sq):
    height_s, width_s = cand0.shape
    n_sites = sites.shape[0]
    npix = height_s * width_s

    width_f = jnp.asarray(width, dtype=jnp.float32)
    height_f = jnp.asarray(height, dtype=jnp.float32)
    scale_f = jnp.asarray(inv_scale_sq, dtype=jnp.float32)
    params = jnp.concatenate([
        jnp.broadcast_to(1.0 / width_f, (L,)),
        jnp.broadcast_to(1.0 / height_f, (L,)),
        jnp.broadcast_to(scale_f, (L,)),
    ]).astype(jnp.float32)

    def tile_order(c):
        c4 = c.reshape(height_s // 8, 8, width_s // 128, 128)
        return c4.transpose(0, 2, 1, 3).reshape(npix)

    r16 = lax.bitcast_convert_type(
        sites[:, 2].astype(jnp.bfloat16), jnp.uint16).astype(jnp.uint32)
    g16 = lax.bitcast_convert_type(
        sites[:, 3].astype(jnp.bfloat16), jnp.uint16).astype(jnp.uint32)
    rg = lax.bitcast_convert_type(r16 | (g16 << 16), jnp.float32)
    sites_cols = jnp.concatenate(
        [sites[:, 0], sites[:, 1], rg, sites[:, 4]])
    sad_sc = _build_sc_kernel(n_sites, npix, width_s)
    out_flat = sad_sc(sites_cols, tile_order(cand0), tile_order(cand1),
                      params)
    out5 = out_flat.reshape(3, height_s // 8, width_s // 128, 8, 128)
    return out5.transpose(1, 3, 2, 4, 0).reshape(height_s, width_s, 3)


# speedup vs baseline: 1.0817x; 1.0034x over previous
import functools

import jax
import jax.numpy as jnp
from jax import lax
from jax.experimental import pallas as pl
from jax.experimental.pallas import tpu as pltpu
from jax.experimental.pallas import tpu_sc as plsc

N_CORES = 2
N_SUBCORES = 16
NW = N_CORES * N_SUBCORES
L = 16
CHUNK = 4096


def _build_sc_kernel(n_sites, npix, width_s):
    mesh = plsc.VectorSubcoreMesh(
        core_axis_name="c", subcore_axis_name="s",
        num_cores=N_CORES, num_subcores=N_SUBCORES)
    per_w = npix // NW
    n_chunks = per_w // CHUNK
    groups = CHUNK // L
    assert width_s & (width_s - 1) == 0 and width_s % 128 == 0
    tr_shift = (8 * width_s).bit_length() - 1

    @functools.partial(
        pl.kernel,
        out_type=jax.ShapeDtypeStruct((npix * 3,), jnp.float32),
        mesh=mesh,
        scratch_types=[
            pltpu.VMEM((n_sites * 4,), jnp.float32),
            [pltpu.VMEM((CHUNK,), jnp.int32) for _ in range(2)],
            [pltpu.VMEM((CHUNK,), jnp.int32) for _ in range(2)],
            [[pltpu.VMEM((CHUNK,), jnp.float32) for _ in range(3)]
             for _ in range(2)],
            pltpu.VMEM((3 * L,), jnp.float32),
            pltpu.VMEM((width_s,), jnp.float32),
            pltpu.VMEM((npix // width_s,), jnp.float32),
            pltpu.SemaphoreType.DMA,
            [pltpu.SemaphoreType.DMA for _ in range(2)],
            [pltpu.SemaphoreType.DMA for _ in range(2)],
        ],
        compiler_params=pltpu.CompilerParams(use_tc_tiling_on_sc=False,
                                             needs_layout_passes=False),
    )
    def sad_sc(sites_hbm, c0_hbm, c1_hbm, par_hbm, out_hbm,
               table_v, c0_v, c1_v, out_v, par_v, lut_u, lut_v,
               tab_sem, in_sems, out_sems):
        wid = lax.axis_index("s") * N_CORES + lax.axis_index("c")
        base_w = wid * per_w

        def start_in(ci):
            b = ci % 2
            base = base_w + ci * CHUNK
            h0 = pltpu.async_copy(c0_hbm.at[pl.ds(base, CHUNK)], c0_v[b],
                                  in_sems[b])
            h1 = pltpu.async_copy(c1_hbm.at[pl.ds(base, CHUNK)], c1_v[b],
                                  in_sems[b])
            return (h0, h1)

        tab_h = pltpu.async_copy(sites_hbm, table_v, tab_sem)
        in_h = [None] * n_chunks
        out_h = [None] * n_chunks
        in_h[0] = start_in(0)
        pltpu.sync_copy(par_hbm, par_v)
        inv_w = par_v[pl.ds(0, L)]
        inv_h = par_v[pl.ds(L, L)]
        scale = par_v[pl.ds(2 * L, L)]
        iota = lax.iota(jnp.int32, L)

        @plsc.parallel_loop(0, width_s // L, 1, unroll=4)
        def bu(i):
            xv = (i * L + iota).astype(jnp.float32)
            lut_u[pl.ds(i * L, L)] = (xv + 0.5) * inv_w * 2.0

        @plsc.parallel_loop(0, npix // width_s // L, 1, unroll=4)
        def bv(i):
            yv = (i * L + iota).astype(jnp.float32)
            lut_v[pl.ds(i * L, L)] = (yv + 0.5) * inv_h * 2.0

        tab_h.wait()

        for ci in range(n_chunks):
            b = ci % 2
            base = base_w + ci * CHUNK
            for h in in_h[ci]:
                h.wait()
            if ci + 1 < n_chunks:
                in_h[ci + 1] = start_in(ci + 1)
            if ci >= 2:
                for h in out_h[ci - 2]:
                    h.wait()
            c0b, c1b, outb = c0_v[b], c1_v[b], out_v[b]

            @plsc.parallel_loop(0, groups, 1, unroll=4)
            def grp(g):
                off = g * L
                w_off = base + off
                wo = w_off & (8 * width_s - 1)
                cc = lax.shift_right_logical(wo, 10)
                r = lax.shift_right_logical(wo, 7) & 7
                l0 = wo & 127
                t_glob = lax.shift_right_logical(w_off, tr_shift)
                idx0 = c0b[pl.ds(off, L)]
                idx1 = c1b[pl.ds(off, L)]
                x0 = plsc.load_gather(table_v, [idx0])
                y0 = plsc.load_gather(table_v, [idx0 + n_sites])
                x1 = plsc.load_gather(table_v, [idx1])
                y1 = plsc.load_gather(table_v, [idx1 + n_sites])
                px2 = lut_u[pl.ds(cc * 128 + l0, L)]
                yidx = jnp.full((L,), t_glob * 8 + r, jnp.int32)
                py2 = plsc.load_gather(lut_v, [yidx])
                t = ((x0 - x1) * (px2 - x0 - x1)
                     + (y0 - y1) * (py2 - y0 - y1)) * scale
                w = 1.0 / (1.0 + jnp.exp(-t))
                rg0 = plsc.load_gather(table_v, [idx0 + 2 * n_sites])
                rg1 = plsc.load_gather(table_v, [idx1 + 2 * n_sites])
                b0 = plsc.load_gather(table_v, [idx0 + 3 * n_sites])
                b1 = plsc.load_gather(table_v, [idx1 + 3 * n_sites])
                wpair = plsc.pack(w, w, format=plsc.PackFormat.INTERLEAVED)
                bf0 = plsc.bitcast(rg0, jnp.bfloat16)
                bf1 = plsc.bitcast(rg1, jnp.bfloat16)
                obf = bf1 + wpair * (bf0 - bf1)
                ou = plsc.bitcast(obf, jnp.uint32)
                himask = jnp.full((L,), 0xFFFF0000, jnp.uint32)
                outb[0][pl.ds(off, L)] = plsc.bitcast(ou << 16, jnp.float32)
                outb[1][pl.ds(off, L)] = plsc.bitcast(ou & himask,
                                                      jnp.float32)
                outb[2][pl.ds(off, L)] = b1 + w * (b0 - b1)

            out_h[ci] = tuple(
                pltpu.async_copy(outb[c],
                                 out_hbm.at[pl.ds(c * npix + base, CHUNK)],
                                 out_sems[b])
                for c in range(3))

        for ci in (n_chunks - 2, n_chunks - 1):
            for h in out_h[ci]:
                h.wait()

    return sad_sc


def kernel(sites, cand0, cand1, width, height, inv_scale_sq):
    height_s, width_s = cand0.shape
    n_sites = sites.shape[0]
    npix = height_s * width_s

    width_f = jnp.asarray(width, dtype=jnp.float32)
    height_f = jnp.asarray(height, dtype=jnp.float32)
    scale_f = jnp.asarray(inv_scale_sq, dtype=jnp.float32)
    params = jnp.concatenate([
        jnp.broadcast_to(1.0 / width_f, (L,)),
        jnp.broadcast_to(1.0 / height_f, (L,)),
        jnp.broadcast_to(scale_f, (L,)),
    ]).astype(jnp.float32)

    def tile_order(c):
        c4 = c.reshape(height_s // 8, 8, width_s // 128, 128)
        return c4.transpose(0, 2, 1, 3).reshape(npix)

    r16 = lax.bitcast_convert_type(
        sites[:, 2].astype(jnp.bfloat16), jnp.uint16).astype(jnp.uint32)
    g16 = lax.bitcast_convert_type(
        sites[:, 3].astype(jnp.bfloat16), jnp.uint16).astype(jnp.uint32)
    rg = lax.bitcast_convert_type(r16 | (g16 << 16), jnp.float32)
    sites_cols = jnp.concatenate(
        [sites[:, 0], sites[:, 1], rg, sites[:, 4]])
    sad_sc = _build_sc_kernel(n_sites, npix, width_s)
    out_flat = sad_sc(sites_cols, tile_order(cand0), tile_order(cand1),
                      params)
    out5 = out_flat.reshape(3, height_s // 8, width_s // 128, 8, 128)
    return out5.transpose(1, 3, 2, 4, 0).reshape(height_s, width_s, 3)
